# baseline (speedup 1.0000x reference)
import functools

import jax
import jax.numpy as jnp
from jax import lax
from jax.experimental import pallas as pl
from jax.experimental.pallas import tpu as pltpu
from jax.experimental.pallas import tpu_sc as plsc

EMBED = 64
MAX_ATOMS = 15
NUM_AT = 6
NUM_AP = 11
NUM_COMB = NUM_AT * NUM_AP
NROW = NUM_COMB + 1
G = 16
CHUNK = 32
SUB = CHUNK // G
OUTW = MAX_ATOMS * EMBED
RFW = CHUNK * EMBED
ATW = CHUNK * 16
MW = 128
OW = CHUNK * OUTW

_GDN = lax.GatherDimensionNumbers(
    offset_dims=(), collapsed_slice_dims=(0,), start_index_map=(0,))


def _splat(v, lane):
    idx = jnp.full((16, 1), lane, jnp.int32)
    return lax.gather(v, idx, _GDN, slice_sizes=(1,),
                      mode=lax.GatherScatterMode.PROMISE_IN_BOUNDS)


def _make_sc_call(R):
    info = plsc.get_sparse_core_info()
    NC, NS = info.num_cores, info.num_subcores
    NW = NC * NS
    per_w = R // NW
    n_chunks = per_w // CHUNK
    n_pairs = n_chunks // 2
    mesh = plsc.VectorSubcoreMesh(core_axis_name="c", subcore_axis_name="s")

    @functools.partial(
        pl.kernel,
        mesh=mesh,
        compiler_params=pltpu.CompilerParams(needs_layout_passes=False),
        out_type=jax.ShapeDtypeStruct((R * OUTW,), jnp.float32),
        scratch_types=[
            pltpu.VMEM((NROW * 16,), jnp.float32),
            pltpu.VMEM((NROW * 16,), jnp.float32),
            pltpu.VMEM((NROW * 16,), jnp.float32),
            pltpu.VMEM((NROW * 16,), jnp.float32),
            pltpu.VMEM(((NUM_AT + NUM_AP) * EMBED,), jnp.float32),
            pltpu.VMEM((2 * RFW,), jnp.float32),
            pltpu.VMEM((2 * ATW,), jnp.int32),
            pltpu.VMEM((2 * ATW,), jnp.int32),
            pltpu.VMEM((2 * MW,), jnp.float32),
            pltpu.VMEM((2 * OW,), jnp.float32),
            pltpu.SemaphoreType.DMA,
            pltpu.SemaphoreType.DMA,
            pltpu.SemaphoreType.DMA,
            pltpu.SemaphoreType.DMA,
        ],
    )
    def sc_call(rf_hbm, at_hbm, ap_hbm, m_hbm, atab_hbm, ptab_hbm, out_hbm,
                comb0, comb1, comb2, comb3, tstage,
                rf_v, at_v, ap_v, m_v, out_v, isem0, isem1, osem0, osem1):
        combs = (comb0, comb1, comb2, comb3)
        isems = (isem0, isem1)
        osems = (osem0, osem1)
        wid = lax.axis_index("s") * NC + lax.axis_index("c")
        base = wid * per_w

        pltpu.sync_copy(atab_hbm, tstage.at[pl.ds(0, NUM_AT * EMBED)])
        pltpu.sync_copy(
            ptab_hbm, tstage.at[pl.ds(NUM_AT * EMBED, NUM_AP * EMBED)])
        arow = [[tstage[pl.ds(a * EMBED + c * 16, 16)] for c in range(4)]
                for a in range(NUM_AT)]
        prow = [[tstage[pl.ds((NUM_AT + p) * EMBED + c * 16, 16)]
                 for c in range(4)] for p in range(NUM_AP)]
        for a in range(NUM_AT):
            for p in range(NUM_AP):
                row = a * NUM_AP + p
                for c in range(4):
                    combs[c][pl.ds(row * 16, 16)] = arow[a][c] + prow[p][c]
        for c in range(4):
            combs[c][pl.ds(NUM_COMB * 16, 16)] = jnp.zeros((16,), jnp.float32)

        iota = lax.iota(jnp.int32, 16)

        def start_in(k, b):
            r0 = base + k * CHUNK
            pltpu.async_copy(rf_hbm.at[pl.ds(r0 * EMBED, RFW)],
                             rf_v.at[pl.ds(b * RFW, RFW)], isems[b])
            pltpu.async_copy(at_hbm.at[pl.ds(r0 * 16, ATW)],
                             at_v.at[pl.ds(b * ATW, ATW)], isems[b])
            pltpu.async_copy(ap_hbm.at[pl.ds(r0 * 16, ATW)],
                             ap_v.at[pl.ds(b * ATW, ATW)], isems[b])
            pltpu.async_copy(m_hbm.at[pl.ds(r0, CHUNK)],
                             m_v.at[pl.ds(b * MW, CHUNK)], isems[b])

        def wait_in(b):
            pltpu.make_async_copy(rf_hbm.at[pl.ds(0, RFW)],
                                  rf_v.at[pl.ds(b * RFW, RFW)],
                                  isems[b]).wait()
            pltpu.make_async_copy(at_hbm.at[pl.ds(0, ATW)],
                                  at_v.at[pl.ds(b * ATW, ATW)],
                                  isems[b]).wait()
            pltpu.make_async_copy(ap_hbm.at[pl.ds(0, ATW)],
                                  ap_v.at[pl.ds(b * ATW, ATW)],
                                  isems[b]).wait()
            pltpu.make_async_copy(m_hbm.at[pl.ds(0, CHUNK)],
                                  m_v.at[pl.ds(b * MW, CHUNK)],
                                  isems[b]).wait()

        def start_out(k, b):
            r0 = base + k * CHUNK
            pltpu.async_copy(out_v.at[pl.ds(b * OW, OW)],
                             out_hbm.at[pl.ds(r0 * OUTW, OW)], osems[b])

        def wait_out(b):
            pltpu.make_async_copy(out_v.at[pl.ds(b * OW, OW)],
                                  out_hbm.at[pl.ds(0, OW)], osems[b]).wait()

        def compute(b):
            def sub_body(sg, _):
                qat = b * ATW + sg * (G * 16)
                qrf = b * RFW + sg * (G * EMBED)
                qout = b * OW + sg * (G * OUTW)
                mgv = m_v[pl.ds(b * MW + sg * G, 16)]
                for l in range(G):
                    atv = at_v[pl.ds(qat + l * 16, 16)]
                    apv = ap_v[pl.ds(qat + l * 16, 16)]
                    mvf = _splat(mgv, l)
                    civ16 = jnp.where(
                        mvf != 0.0, atv * NUM_AP + apv,
                        jnp.full((16,), NUM_COMB, jnp.int32)) * 16
                    rfm = [rf_v[pl.ds(qrf + l * EMBED + c * 16, 16)] * mvf
                           for c in range(4)]
                    obase = qout + l * OUTW
                    for j in range(MAX_ATOMS):
                        idx = _splat(civ16, j) + iota
                        for c in range(4):
                            out_v[pl.ds(obase + j * EMBED + c * 16, 16)] = (
                                plsc.load_gather(combs[c], [idx]) + rfm[c])
                return 0

            lax.fori_loop(0, SUB, sub_body, 0)

        start_in(0, 0)
        start_in(1, 1)

        def pair_body(kp, _):
            for b in range(2):
                k = kp * 2 + b
                wait_in(b)

                @pl.when(kp > 0)
                def _():
                    wait_out(b)

                compute(b)
                start_out(k, b)

                @pl.when(k + 2 < n_chunks)
                def _():
                    start_in(k + 2, b)
            return 0

        lax.fori_loop(0, n_pairs, pair_body, 0)
        wait_out(0)
        wait_out(1)

    return sc_call


def kernel(res_feat, atom_types, atom_positions, mask_atoms, block_lengths,
           atom_table, pos_table):
    N, L, E = res_feat.shape
    A = atom_types.shape[-1]
    R = N * L
    rf1 = res_feat.reshape(R * E)
    at1 = atom_types.reshape(R * A).astype(jnp.int32)
    ap1 = atom_positions.reshape(R * A).astype(jnp.int32)
    mf = mask_atoms[:, :, 1].reshape(R).astype(jnp.float32)
    atab1 = atom_table.reshape(NUM_AT * EMBED)
    ptab1 = pos_table.reshape(NUM_AP * EMBED)
    sc_call = _make_sc_call(R)
    out = sc_call(rf1, at1, ap1, mf, atab1, ptab1)
    return out.reshape(R * MAX_ATOMS, EMBED)


# speedup vs baseline: 1.8926x; 1.8926x over previous
import functools

import jax
import jax.numpy as jnp
from jax import lax
from jax.experimental import pallas as pl
from jax.experimental.pallas import tpu as pltpu
from jax.experimental.pallas import tpu_sc as plsc

EMBED = 64
MAX_ATOMS = 15
NUM_AT = 6
NUM_AP = 11
NUM_COMB = NUM_AT * NUM_AP
NROW = NUM_COMB + 1
G = 16
CHUNK = 32
SUB = CHUNK // G
OUTW = MAX_ATOMS * EMBED
RFW = CHUNK * EMBED
ATW = CHUNK * 16
MW = 128
OW = CHUNK * OUTW

_GDN = lax.GatherDimensionNumbers(
    offset_dims=(), collapsed_slice_dims=(0,), start_index_map=(0,))


def _splat(v, lane):
    idx = jnp.full((16, 1), lane, jnp.int32)
    return lax.gather(v, idx, _GDN, slice_sizes=(1,),
                      mode=lax.GatherScatterMode.PROMISE_IN_BOUNDS)


def _make_sc_call(R):
    info = plsc.get_sparse_core_info()
    NC, NS = info.num_cores, info.num_subcores
    NW = NC * NS
    per_w = R // NW
    n_chunks = per_w // CHUNK
    n_pairs = n_chunks // 2
    mesh = plsc.VectorSubcoreMesh(core_axis_name="c", subcore_axis_name="s")

    @functools.partial(
        pl.kernel,
        mesh=mesh,
        compiler_params=pltpu.CompilerParams(needs_layout_passes=False),
        out_type=jax.ShapeDtypeStruct((R * OUTW,), jnp.float32),
        scratch_types=[
            pltpu.VMEM((NROW * 16,), jnp.float32),
            pltpu.VMEM((NROW * 16,), jnp.float32),
            pltpu.VMEM((NROW * 16,), jnp.float32),
            pltpu.VMEM((NROW * 16,), jnp.float32),
            pltpu.VMEM(((NUM_AT + NUM_AP) * EMBED,), jnp.float32),
            pltpu.VMEM((2 * RFW,), jnp.float32),
            pltpu.VMEM((2 * ATW,), jnp.int32),
            pltpu.VMEM((2 * ATW,), jnp.int32),
            pltpu.VMEM((2 * MW,), jnp.float32),
            pltpu.VMEM((2 * OW,), jnp.float32),
            pltpu.SemaphoreType.DMA,
            pltpu.SemaphoreType.DMA,
            pltpu.SemaphoreType.DMA,
            pltpu.SemaphoreType.DMA,
        ],
    )
    def sc_call(rf_hbm, at_hbm, ap_hbm, m_hbm, atab_hbm, ptab_hbm, out_hbm,
                comb0, comb1, comb2, comb3, tstage,
                rf_v, at_v, ap_v, m_v, out_v, isem0, isem1, osem0, osem1):
        combs = (comb0, comb1, comb2, comb3)
        isems = (isem0, isem1)
        osems = (osem0, osem1)
        wid = lax.axis_index("s") * NC + lax.axis_index("c")
        base = wid * per_w

        pltpu.sync_copy(atab_hbm, tstage.at[pl.ds(0, NUM_AT * EMBED)])
        pltpu.sync_copy(
            ptab_hbm, tstage.at[pl.ds(NUM_AT * EMBED, NUM_AP * EMBED)])
        arow = [[tstage[pl.ds(a * EMBED + c * 16, 16)] for c in range(4)]
                for a in range(NUM_AT)]
        prow = [[tstage[pl.ds((NUM_AT + p) * EMBED + c * 16, 16)]
                 for c in range(4)] for p in range(NUM_AP)]
        for a in range(NUM_AT):
            for p in range(NUM_AP):
                row = a * NUM_AP + p
                for c in range(4):
                    combs[c][pl.ds(row * 16, 16)] = arow[a][c] + prow[p][c]
        for c in range(4):
            combs[c][pl.ds(NUM_COMB * 16, 16)] = jnp.zeros((16,), jnp.float32)

        iota = lax.iota(jnp.int32, 16)

        def start_in(k, b):
            r0 = base + k * CHUNK
            pltpu.async_copy(rf_hbm.at[pl.ds(r0 * EMBED, RFW)],
                             rf_v.at[pl.ds(b * RFW, RFW)], isems[b])
            pltpu.async_copy(at_hbm.at[pl.ds(r0 * 16, ATW)],
                             at_v.at[pl.ds(b * ATW, ATW)], isems[b])
            pltpu.async_copy(ap_hbm.at[pl.ds(r0 * 16, ATW)],
                             ap_v.at[pl.ds(b * ATW, ATW)], isems[b])
            pltpu.async_copy(m_hbm.at[pl.ds(r0, CHUNK)],
                             m_v.at[pl.ds(b * MW, CHUNK)], isems[b])

        def wait_in(b):
            pltpu.make_async_copy(rf_hbm.at[pl.ds(0, RFW)],
                                  rf_v.at[pl.ds(b * RFW, RFW)],
                                  isems[b]).wait()
            pltpu.make_async_copy(at_hbm.at[pl.ds(0, ATW)],
                                  at_v.at[pl.ds(b * ATW, ATW)],
                                  isems[b]).wait()
            pltpu.make_async_copy(ap_hbm.at[pl.ds(0, ATW)],
                                  ap_v.at[pl.ds(b * ATW, ATW)],
                                  isems[b]).wait()
            pltpu.make_async_copy(m_hbm.at[pl.ds(0, CHUNK)],
                                  m_v.at[pl.ds(b * MW, CHUNK)],
                                  isems[b]).wait()

        def start_out(k, b):
            r0 = base + k * CHUNK
            pltpu.async_copy(out_v.at[pl.ds(b * OW, OW)],
                             out_hbm.at[pl.ds(r0 * OUTW, OW)], osems[b])

        def wait_out(b):
            pltpu.make_async_copy(out_v.at[pl.ds(b * OW, OW)],
                                  out_hbm.at[pl.ds(0, OW)], osems[b]).wait()

        def compute(b):
            @plsc.parallel_loop(0, CHUNK, unroll=4)
            def _body(r):
                mv16 = m_v[pl.ds(b * MW + r, 16)]
                mvf = _splat(mv16, 0)
                atv = at_v[pl.ds(b * ATW + r * 16, 16)]
                apv = ap_v[pl.ds(b * ATW + r * 16, 16)]
                civ16 = jnp.where(
                    mvf != 0.0, atv * NUM_AP + apv,
                    jnp.full((16,), NUM_COMB, jnp.int32)) * 16
                rfm = [rf_v[pl.ds(b * RFW + r * EMBED + c * 16, 16)] * mvf
                       for c in range(4)]
                obase = b * OW + r * OUTW
                for j in range(MAX_ATOMS):
                    idx = _splat(civ16, j) + iota
                    for c in range(4):
                        out_v[pl.ds(obase + j * EMBED + c * 16, 16)] = (
                            plsc.load_gather(combs[c], [idx]) + rfm[c])

        start_in(0, 0)
        start_in(1, 1)

        def pair_body(kp, _):
            for b in range(2):
                k = kp * 2 + b
                wait_in(b)

                @pl.when(kp > 0)
                def _():
                    wait_out(b)

                compute(b)
                start_out(k, b)

                @pl.when(k + 2 < n_chunks)
                def _():
                    start_in(k + 2, b)
            return 0

        lax.fori_loop(0, n_pairs, pair_body, 0)
        wait_out(0)
        wait_out(1)

    return sc_call


def kernel(res_feat, atom_types, atom_positions, mask_atoms, block_lengths,
           atom_table, pos_table):
    N, L, E = res_feat.shape
    A = atom_types.shape[-1]
    R = N * L
    rf1 = res_feat.reshape(R * E)
    at1 = atom_types.reshape(R * A).astype(jnp.int32)
    ap1 = atom_positions.reshape(R * A).astype(jnp.int32)
    mf = mask_atoms[:, :, 1].reshape(R).astype(jnp.float32)
    atab1 = atom_table.reshape(NUM_AT * EMBED)
    ptab1 = pos_table.reshape(NUM_AP * EMBED)
    sc_call = _make_sc_call(R)
    out = sc_call(rf1, at1, ap1, mf, atab1, ptab1)
    return out.reshape(R * MAX_ATOMS, EMBED)


# speedup vs baseline: 1.9065x; 1.0073x over previous
import functools

import jax
import jax.numpy as jnp
from jax import lax
from jax.experimental import pallas as pl
from jax.experimental.pallas import tpu as pltpu
from jax.experimental.pallas import tpu_sc as plsc

EMBED = 64
MAX_ATOMS = 15
NUM_AT = 6
NUM_AP = 11
NUM_COMB = NUM_AT * NUM_AP
NROW = NUM_COMB + 1
G = 16
CHUNK = 32
SUB = CHUNK // G
OUTW = MAX_ATOMS * EMBED
RFW = CHUNK * EMBED
ATW = CHUNK * 16
MW = 128
OW = CHUNK * OUTW

_GDN = lax.GatherDimensionNumbers(
    offset_dims=(), collapsed_slice_dims=(0,), start_index_map=(0,))


def _splat(v, lane):
    idx = jnp.full((16, 1), lane, jnp.int32)
    return lax.gather(v, idx, _GDN, slice_sizes=(1,),
                      mode=lax.GatherScatterMode.PROMISE_IN_BOUNDS)


def _make_sc_call(R):
    info = plsc.get_sparse_core_info()
    NC, NS = info.num_cores, info.num_subcores
    NW = NC * NS
    per_w = R // NW
    n_chunks = per_w // CHUNK
    n_pairs = n_chunks // 2
    mesh = plsc.VectorSubcoreMesh(core_axis_name="c", subcore_axis_name="s")

    @functools.partial(
        pl.kernel,
        mesh=mesh,
        compiler_params=pltpu.CompilerParams(needs_layout_passes=False),
        out_type=jax.ShapeDtypeStruct((R * OUTW,), jnp.float32),
        scratch_types=[
            pltpu.VMEM((NROW * 16,), jnp.float32),
            pltpu.VMEM((NROW * 16,), jnp.float32),
            pltpu.VMEM((NROW * 16,), jnp.float32),
            pltpu.VMEM((NROW * 16,), jnp.float32),
            pltpu.VMEM(((NUM_AT + NUM_AP) * EMBED,), jnp.float32),
            pltpu.VMEM((2 * RFW,), jnp.float32),
            pltpu.VMEM((2 * ATW,), jnp.int32),
            pltpu.VMEM((2 * ATW,), jnp.int32),
            pltpu.VMEM((2 * MW,), jnp.float32),
            pltpu.VMEM((2 * OW,), jnp.float32),
            pltpu.SemaphoreType.DMA,
            pltpu.SemaphoreType.DMA,
            pltpu.SemaphoreType.DMA,
            pltpu.SemaphoreType.DMA,
        ],
    )
    def sc_call(rf_hbm, at_hbm, ap_hbm, m_hbm, atab_hbm, ptab_hbm, out_hbm,
                comb0, comb1, comb2, comb3, tstage,
                rf_v, at_v, ap_v, m_v, out_v, isem0, isem1, osem0, osem1):
        combs = (comb0, comb1, comb2, comb3)
        isems = (isem0, isem1)
        osems = (osem0, osem1)
        wid = lax.axis_index("s") * NC + lax.axis_index("c")
        base = wid * per_w

        pltpu.sync_copy(atab_hbm, tstage.at[pl.ds(0, NUM_AT * EMBED)])
        pltpu.sync_copy(
            ptab_hbm, tstage.at[pl.ds(NUM_AT * EMBED, NUM_AP * EMBED)])
        arow = [[tstage[pl.ds(a * EMBED + c * 16, 16)] for c in range(4)]
                for a in range(NUM_AT)]
        prow = [[tstage[pl.ds((NUM_AT + p) * EMBED + c * 16, 16)]
                 for c in range(4)] for p in range(NUM_AP)]
        for a in range(NUM_AT):
            for p in range(NUM_AP):
                row = a * NUM_AP + p
                for c in range(4):
                    combs[c][pl.ds(row * 16, 16)] = arow[a][c] + prow[p][c]
        for c in range(4):
            combs[c][pl.ds(NUM_COMB * 16, 16)] = jnp.zeros((16,), jnp.float32)

        iota = lax.iota(jnp.int32, 16)

        def start_in(k, b):
            r0 = base + k * CHUNK
            pltpu.async_copy(rf_hbm.at[pl.ds(r0 * EMBED, RFW)],
                             rf_v.at[pl.ds(b * RFW, RFW)], isems[b])
            pltpu.async_copy(at_hbm.at[pl.ds(r0 * 16, ATW)],
                             at_v.at[pl.ds(b * ATW, ATW)], isems[b])
            pltpu.async_copy(ap_hbm.at[pl.ds(r0 * 16, ATW)],
                             ap_v.at[pl.ds(b * ATW, ATW)], isems[b])
            pltpu.async_copy(m_hbm.at[pl.ds(r0, CHUNK)],
                             m_v.at[pl.ds(b * MW, CHUNK)], isems[b])

        def wait_in(b):
            pltpu.make_async_copy(rf_hbm.at[pl.ds(0, RFW)],
                                  rf_v.at[pl.ds(b * RFW, RFW)],
                                  isems[b]).wait()
            pltpu.make_async_copy(at_hbm.at[pl.ds(0, ATW)],
                                  at_v.at[pl.ds(b * ATW, ATW)],
                                  isems[b]).wait()
            pltpu.make_async_copy(ap_hbm.at[pl.ds(0, ATW)],
                                  ap_v.at[pl.ds(b * ATW, ATW)],
                                  isems[b]).wait()
            pltpu.make_async_copy(m_hbm.at[pl.ds(0, CHUNK)],
                                  m_v.at[pl.ds(b * MW, CHUNK)],
                                  isems[b]).wait()

        def start_out(k, b):
            r0 = base + k * CHUNK
            pltpu.async_copy(out_v.at[pl.ds(b * OW, OW)],
                             out_hbm.at[pl.ds(r0 * OUTW, OW)], osems[b])

        def wait_out(b):
            pltpu.make_async_copy(out_v.at[pl.ds(b * OW, OW)],
                                  out_hbm.at[pl.ds(0, OW)], osems[b]).wait()

        def compute(b):
            @plsc.parallel_loop(0, CHUNK, unroll=8)
            def _body(r):
                mv16 = m_v[pl.ds(b * MW + r, 16)]
                mvf = _splat(mv16, 0)
                atv = at_v[pl.ds(b * ATW + r * 16, 16)]
                apv = ap_v[pl.ds(b * ATW + r * 16, 16)]
                civ16 = jnp.where(
                    mvf != 0.0, atv * NUM_AP + apv,
                    jnp.full((16,), NUM_COMB, jnp.int32)) * 16
                rfm = [rf_v[pl.ds(b * RFW + r * EMBED + c * 16, 16)] * mvf
                       for c in range(4)]
                obase = b * OW + r * OUTW
                for j in range(MAX_ATOMS):
                    idx = _splat(civ16, j) + iota
                    for c in range(4):
                        out_v[pl.ds(obase + j * EMBED + c * 16, 16)] = (
                            plsc.load_gather(combs[c], [idx]) + rfm[c])

        start_in(0, 0)
        start_in(1, 1)

        def pair_body(kp, _):
            for b in range(2):
                k = kp * 2 + b
                wait_in(b)

                @pl.when(kp > 0)
                def _():
                    wait_out(b)

                compute(b)
                start_out(k, b)

                @pl.when(k + 2 < n_chunks)
                def _():
                    start_in(k + 2, b)
            return 0

        lax.fori_loop(0, n_pairs, pair_body, 0)
        wait_out(0)
        wait_out(1)

    return sc_call


def kernel(res_feat, atom_types, atom_positions, mask_atoms, block_lengths,
           atom_table, pos_table):
    N, L, E = res_feat.shape
    A = atom_types.shape[-1]
    R = N * L
    rf1 = res_feat.reshape(R * E)
    at1 = atom_types.reshape(R * A).astype(jnp.int32)
    ap1 = atom_positions.reshape(R * A).astype(jnp.int32)
    mf = mask_atoms[:, :, 1].reshape(R).astype(jnp.float32)
    atab1 = atom_table.reshape(NUM_AT * EMBED)
    ptab1 = pos_table.reshape(NUM_AP * EMBED)
    sc_call = _make_sc_call(R)
    out = sc_call(rf1, at1, ap1, mf, atab1, ptab1)
    return out.reshape(R * MAX_ATOMS, EMBED)
